# baseline (speedup 1.0000x reference)
import functools

import jax
import jax.numpy as jnp
from jax.experimental import pallas as pl
from jax.experimental.pallas import tpu as pltpu

_SEQ = 200
_PAD = 256


def _body(ts_ref, emb_ref, flag_ref, out_ref):
    ts = ts_ref[...]
    emb = emb_ref[...]
    flag = flag_ref[0, 0]

    r = ts.shape[0]
    a = ts[:, None, :]
    b = ts[:, :, None]
    iota_k = jax.lax.broadcasted_iota(jnp.int32, (_PAD, _PAD), 1)
    iota_j = jax.lax.broadcasted_iota(jnp.int32, (_PAD, _PAD), 0)
    kmask = (iota_k < iota_j)[None]
    cmp = (a < b) | ((a == b) & kmask)
    rank = jnp.sum(cmp.astype(jnp.int32), axis=2)

    i_iota = jax.lax.broadcasted_iota(jnp.int32, (1, _SEQ, _PAD), 1)
    j_iota = jax.lax.broadcasted_iota(jnp.int32, (1, _SEQ, _PAD), 2)
    oh = (rank[:, None, :] == i_iota).astype(jnp.float32)
    oh = jnp.where(flag > 0.5, (j_iota == _SEQ).astype(jnp.float32), oh)

    out = jax.lax.dot_general(
        oh, emb,
        dimension_numbers=(((2,), (0,)), ((), ())),
        preferred_element_type=jnp.float32,
        precision=jax.lax.Precision.HIGHEST,
    )
    out_ref[...] = out


@jax.jit
def kernel(ts, pos_emb):
    batch, seq = ts.shape
    assert seq == _SEQ
    ts_pad = jnp.pad(ts, ((0, 0), (0, _PAD - _SEQ)), constant_values=jnp.inf)
    emb_pad = jnp.pad(pos_emb, ((0, _PAD - (_SEQ + 1)), (0, 0)))
    flag = jnp.all(ts == 0.0).astype(jnp.float32).reshape(1, 1)

    r = 16
    grid = (batch // r,)
    out = pl.pallas_call(
        _body,
        grid=grid,
        in_specs=[
            pl.BlockSpec((r, _PAD), lambda i: (i, 0)),
            pl.BlockSpec((_PAD, 64), lambda i: (0, 0)),
            pl.BlockSpec(memory_space=pltpu.SMEM),
        ],
        out_specs=pl.BlockSpec((r, _SEQ, 64), lambda i: (i, 0, 0)),
        out_shape=jax.ShapeDtypeStruct((batch, _SEQ, 64), jnp.float32),
    )(ts_pad, emb_pad, flag)
    return out


# speedup vs baseline: 21.5278x; 21.5278x over previous
import functools

import jax
import jax.numpy as jnp
from jax import lax
from jax.experimental import pallas as pl
from jax.experimental.pallas import tpu as pltpu
from jax.experimental.pallas import tpu_sc as plsc

_SEQ = 200
_D = 64
_L = 16
_V = 16
_NC = 2
_NS = 16
_NW = _NC * _NS
_TPAD = 256


def _cmp_exchange(ka, va, kb, vb, asc):
    m = (ka <= kb) if asc else (ka >= kb)
    kl = jnp.where(m, ka, kb)
    vl = jnp.where(m, va, vb)
    kh = jnp.where(m, kb, ka)
    vh = jnp.where(m, vb, va)
    return kl, vl, kh, vh


def _bitonic_sort(keys, vals):
    for v in range(_V):
        keys[v], vals[v] = plsc.sort_key_val(keys[v], vals[v],
                                             descending=bool(v & 1))
    for vk in (2, 4, 8, 16):
        vj = vk // 2
        while vj >= 1:
            for v in range(_V):
                if (v & vj) == 0:
                    p = v | vj
                    asc = (v & vk) == 0
                    keys[v], vals[v], keys[p], vals[p] = _cmp_exchange(
                        keys[v], vals[v], keys[p], vals[p], asc)
            vj //= 2
        for v in range(_V):
            asc = (v & vk) == 0
            keys[v], vals[v] = plsc.sort_key_val(keys[v], vals[v],
                                                 descending=not asc)
    return keys, vals


def _row_order(ts_buf, row_base, iota, idxm1, flag_v):
    inf = jnp.float32(jnp.inf)
    keys = []
    vals = []
    for g in range(_V):
        if g < 12:
            k = ts_buf[pl.ds(row_base + g * _L, _L)]
        elif g == 12:
            k = ts_buf[pl.ds(row_base + 12 * _L, _L)]
            k = jnp.where(iota < 8, k, inf)
        else:
            k = jnp.full((_L,), inf, jnp.float32)
        keys.append(k)
        vals.append(iota + g * _L)

    keys, vals = _bitonic_sort(keys, vals)

    comp = []
    carry_seg = jnp.int32(0)
    prev_last = jnp.float32(-jnp.inf)
    for g in range(_V):
        shifted = keys[g].at[idxm1].get(mode="promise_in_bounds")
        prev = jnp.where(iota == 0, prev_last, shifted)
        nr = keys[g] != prev
        if g == 0:
            nr = nr | (iota == 0)
        cand = jnp.where(nr, iota + g * _L, 0)
        seg = plsc.cummax(jnp.maximum(cand, carry_seg))
        carry_seg = jnp.max(seg)
        prev_last = jnp.max(keys[g])
        comp.append(seg * 256 + vals[g])

    comp, vals = _bitonic_sort(comp, vals)

    for g in range(_V):
        vals[g] = jnp.where(flag_v > 0, 200, vals[g])
    return vals


def _sc_body(ts_ref, table_ref, flag_ref, out_ref,
             ts_buf, order_lo, order_hi, rows_buf, flag_buf, sem_g, sem_s):
    wid = lax.axis_index("s") * _NC + lax.axis_index("c")
    rows_per_tile = 128
    base = wid * rows_per_tile

    pltpu.sync_copy(ts_ref.at[pl.ds(base * _SEQ, rows_per_tile * _SEQ)],
                    ts_buf.at[pl.ds(0, rows_per_tile * _SEQ)])
    pltpu.sync_copy(flag_ref, flag_buf)

    iota = lax.iota(jnp.int32, _L)
    idxm1 = jnp.maximum(iota - 1, 0)
    flag_v = flag_buf[...]

    def body(r, _):
        vals = _row_order(ts_buf, r * _SEQ, iota, idxm1, flag_v)

        @pl.when(r != 0)
        def _():
            pltpu.make_async_copy(
                rows_buf.at[pl.ds(0, _SEQ)], out_ref.at[base], sem_s).wait()

        for g in range(8):
            order_lo[pl.ds(g * _L, _L)] = vals[g]
        for g in range(8, 13):
            order_hi[pl.ds((g - 8) * _L, _L)] = vals[g]

        pltpu.async_copy(table_ref.at[order_lo],
                         rows_buf.at[pl.ds(0, 128)], sem_g)
        pltpu.async_copy(table_ref.at[order_hi],
                         rows_buf.at[pl.ds(128, 80)], sem_g)
        pltpu.make_async_copy(table_ref.at[order_lo],
                              rows_buf.at[pl.ds(0, 128)], sem_g).wait()
        pltpu.make_async_copy(table_ref.at[order_hi],
                              rows_buf.at[pl.ds(128, 80)], sem_g).wait()

        pltpu.async_copy(rows_buf.at[pl.ds(0, _SEQ)],
                         out_ref.at[base + r], sem_s)
        return _

    lax.fori_loop(0, rows_per_tile, body, None)
    pltpu.make_async_copy(
        rows_buf.at[pl.ds(0, _SEQ)], out_ref.at[base], sem_s).wait()


@jax.jit
def kernel(ts, pos_emb):
    batch, seq = ts.shape
    assert seq == _SEQ and batch == _NW * 128
    table = jnp.pad(pos_emb, ((0, _TPAD - pos_emb.shape[0]), (0, 0)))
    flag = jnp.full((_L,), jnp.all(ts == 0.0).astype(jnp.int32))
    ts_flat = ts.reshape(-1)

    mesh = plsc.VectorSubcoreMesh(core_axis_name="c", subcore_axis_name="s")
    run = pl.kernel(
        _sc_body,
        out_type=jax.ShapeDtypeStruct((batch, _SEQ, _D), jnp.float32),
        mesh=mesh,
        compiler_params=pltpu.CompilerParams(
            needs_layout_passes=False, use_tc_tiling_on_sc=False),
        scratch_types=[
            pltpu.VMEM((128 * _SEQ + 8,), jnp.float32),
            pltpu.VMEM((128,), jnp.int32),
            pltpu.VMEM((80,), jnp.int32),
            pltpu.VMEM((208, _D), jnp.float32),
            pltpu.VMEM((_L,), jnp.int32),
            pltpu.SemaphoreType.DMA,
            pltpu.SemaphoreType.DMA,
        ],
    )
    return run(ts_flat, table, flag)


# speedup vs baseline: 40.3815x; 1.8758x over previous
import functools

import jax
import jax.numpy as jnp
from jax import lax
from jax.experimental import pallas as pl
from jax.experimental.pallas import tpu as pltpu
from jax.experimental.pallas import tpu_sc as plsc

_SEQ = 200
_D = 64
_L = 16
_V = 16
_NC = 2
_NS = 16
_NW = _NC * _NS
_RPT = 128


def _cmp_exchange(ka, va, kb, vb, asc):
    m = (ka <= kb) if asc else (ka >= kb)
    kl = jnp.where(m, ka, kb)
    vl = jnp.where(m, va, vb)
    kh = jnp.where(m, kb, ka)
    vh = jnp.where(m, vb, va)
    return kl, vl, kh, vh


def _bitonic_sort(keys, vals):
    for v in range(_V):
        keys[v], vals[v] = plsc.sort_key_val(keys[v], vals[v],
                                             descending=bool(v & 1))
    for vk in (2, 4, 8, 16):
        vj = vk // 2
        while vj >= 1:
            for v in range(_V):
                if (v & vj) == 0:
                    p = v | vj
                    asc = (v & vk) == 0
                    keys[v], vals[v], keys[p], vals[p] = _cmp_exchange(
                        keys[v], vals[v], keys[p], vals[p], asc)
            vj //= 2
        for v in range(_V):
            asc = (v & vk) == 0
            keys[v], vals[v] = plsc.sort_key_val(keys[v], vals[v],
                                                 descending=not asc)
    return keys, vals


def _row_order(ts_buf, row_base, iota, idxm1):
    inf = jnp.float32(jnp.inf)
    keys = []
    vals = []
    for g in range(_V):
        if g < 12:
            k = ts_buf[pl.ds(row_base + g * _L, _L)]
        elif g == 12:
            k = ts_buf[pl.ds(row_base + 12 * _L, _L)]
            k = jnp.where(iota < 8, k, inf)
        else:
            k = jnp.full((_L,), inf, jnp.float32)
        keys.append(k)
        vals.append(iota + g * _L)

    keys, vals = _bitonic_sort(keys, vals)

    comp = []
    carry_seg = jnp.int32(0)
    prev_last = jnp.float32(-jnp.inf)
    for g in range(_V):
        shifted = keys[g].at[idxm1].get(mode="promise_in_bounds")
        prev = jnp.where(iota == 0, prev_last, shifted)
        nr = keys[g] != prev
        if g == 0:
            nr = nr | (iota == 0)
        cand = jnp.where(nr, iota + g * _L, 0)
        seg = plsc.cummax(jnp.maximum(cand, carry_seg))
        carry_seg = jnp.max(seg)
        prev_last = jnp.max(keys[g])
        comp.append(seg * 256 + vals[g])

    comp, vals = _bitonic_sort(comp, vals)
    return vals


def _sc_body(ts_ref, table_ref, flag_ref, out_ref,
             ts_buf, table_buf, idx_lo, idx_hi, flag_buf, sem_s):
    wid = lax.axis_index("s") * _NC + lax.axis_index("c")
    base = wid * _RPT

    pltpu.sync_copy(ts_ref.at[pl.ds(base * _SEQ, _RPT * _SEQ)],
                    ts_buf.at[pl.ds(0, _RPT * _SEQ)])
    pltpu.sync_copy(table_ref, table_buf)
    pltpu.sync_copy(flag_ref, flag_buf)

    iota = lax.iota(jnp.int32, _L)
    idxm1 = jnp.maximum(iota - 1, 0)
    src_off = jnp.max(flag_buf[...]) * 256

    def body(r, _):
        b = r & 1
        vals = _row_order(ts_buf, r * _SEQ, iota, idxm1)

        @pl.when(r >= 2)
        def _wait():
            pltpu.make_async_copy(table_buf.at[pl.ds(0, 128)],
                                  out_ref.at[idx_lo.at[0]], sem_s).wait()
            pltpu.make_async_copy(table_buf.at[pl.ds(0, 72)],
                                  out_ref.at[idx_hi.at[0]], sem_s).wait()

        b_vec = jnp.full((_L,), 0, jnp.int32) + b
        rowbase = (base + r) * _SEQ
        for g in range(_V):
            value = iota + (g * _L) + rowbase
            j = vals[g]
            m_lo = j < 128
            m_hi = (j >= 128) & (j < _SEQ)
            plsc.store_scatter(idx_lo, [b_vec, j], value, mask=m_lo)
            plsc.store_scatter(idx_hi, [b_vec, j - 128], value, mask=m_hi)

        pltpu.async_copy(table_buf.at[pl.ds(src_off, 128)],
                         out_ref.at[idx_lo.at[b]], sem_s)
        pltpu.async_copy(table_buf.at[pl.ds(src_off + 128, 72)],
                         out_ref.at[idx_hi.at[b]], sem_s)
        return _

    lax.fori_loop(0, _RPT, body, None)
    for _ in range(2):
        pltpu.make_async_copy(table_buf.at[pl.ds(0, 128)],
                              out_ref.at[idx_lo.at[0]], sem_s).wait()
        pltpu.make_async_copy(table_buf.at[pl.ds(0, 72)],
                              out_ref.at[idx_hi.at[0]], sem_s).wait()


@jax.jit
def kernel(ts, pos_emb):
    batch, seq = ts.shape
    assert seq == _SEQ and batch == _NW * _RPT
    table = jnp.pad(pos_emb, ((0, 256 - pos_emb.shape[0]), (0, 0)))
    zero_blk = jnp.broadcast_to(pos_emb[_SEQ], (256, _D))
    table_aug = jnp.concatenate([table, zero_blk], axis=0)
    flag = jnp.full((_L,), jnp.all(ts == 0.0).astype(jnp.int32))
    ts_flat = ts.reshape(-1)

    mesh = plsc.VectorSubcoreMesh(core_axis_name="c", subcore_axis_name="s")
    run = pl.kernel(
        _sc_body,
        out_type=jax.ShapeDtypeStruct((batch * _SEQ, _D), jnp.float32),
        mesh=mesh,
        compiler_params=pltpu.CompilerParams(
            needs_layout_passes=False, use_tc_tiling_on_sc=False),
        scratch_types=[
            pltpu.VMEM((_RPT * _SEQ + 8,), jnp.float32),
            pltpu.VMEM((512, _D), jnp.float32),
            pltpu.VMEM((2, 128), jnp.int32),
            pltpu.VMEM((2, 72), jnp.int32),
            pltpu.VMEM((_L,), jnp.int32),
            pltpu.SemaphoreType.DMA,
        ],
    )
    out2d = run(ts_flat, table_aug, flag)
    return out2d.reshape(batch, _SEQ, _D)
